# baseline (speedup 1.0000x reference)
import functools

import jax
import jax.numpy as jnp
from jax import lax
from jax.experimental import pallas as pl
from jax.experimental.pallas import tpu as pltpu
from jax.experimental.pallas import tpu_sc as plsc

BATCH = 16384
EMBED_K = 32
L = 16
NW = 32
B_PER_W = BATCH // NW
N_ELEMS = B_PER_W * EMBED_K
RING = 10


def _body(u_hbm, v_hbm, wt_hbm, ht_hbm, out_hbm,
          uidx_s, vidx_s, ustage, vstage,
          uflat, vflat, out_v, sem):
    wid = lax.axis_index("s") * 2 + lax.axis_index("c")
    base = wid * B_PER_W

    wt3 = wt_hbm.reshape(4, 8, 1000000)
    ht3 = ht_hbm.reshape(4, 8, 1000000)

    pltpu.sync_copy(u_hbm.at[pl.ds(base, B_PER_W)], uidx_s.at[pl.ds(0, B_PER_W)])
    pltpu.sync_copy(v_hbm.at[pl.ds(base, B_PER_W)], vidx_s.at[pl.ds(0, B_PER_W)])

    lane = lax.iota(jnp.int32, L)
    tk_a, s_a = lane >> 3, lane & 7
    tk_b, s_b = 2 + (lane >> 3), lane & 7
    ka_base = lane * B_PER_W
    kb_base = (lane + L) * B_PER_W

    def fire(j):
        slot = lax.rem(j, RING)
        u = uidx_s[pl.ds(j, L)][0]
        v = vidx_s[pl.ds(j, L)][0]
        uoff = pl.multiple_of((u >> 7) * 128, 128)
        voff = pl.multiple_of((v >> 7) * 128, 128)
        pltpu.async_copy(
            wt3.at[:, :, pl.ds(uoff, 128)], ustage.at[slot], sem)
        pltpu.async_copy(
            ht3.at[:, :, pl.ds(voff, 128)], vstage.at[slot], sem)

    def drain(j):
        slot = lax.rem(j, RING)
        pltpu.make_async_copy(
            wt3.at[:, :, pl.ds(0, 128)], ustage.at[slot], sem).wait()
        pltpu.make_async_copy(
            ht3.at[:, :, pl.ds(0, 128)], vstage.at[slot], sem).wait()

    def extract(j):
        slot16 = jnp.full((L,), lax.rem(j, RING), jnp.int32)
        ucol = jnp.full((L,), uidx_s[pl.ds(j, L)][0] & 127, jnp.int32)
        vcol = jnp.full((L,), vidx_s[pl.ds(j, L)][0] & 127, jnp.int32)
        ua = plsc.load_gather(ustage, [slot16, tk_a, s_a, ucol])
        ub = plsc.load_gather(ustage, [slot16, tk_b, s_b, ucol])
        va = plsc.load_gather(vstage, [slot16, tk_a, s_a, vcol])
        vb = plsc.load_gather(vstage, [slot16, tk_b, s_b, vcol])
        plsc.store_scatter(uflat, [ka_base + j], ua)
        plsc.store_scatter(uflat, [kb_base + j], ub)
        plsc.store_scatter(vflat, [ka_base + j], va)
        plsc.store_scatter(vflat, [kb_base + j], vb)

    def prime(j, carry):
        fire(j)
        return carry

    lax.fori_loop(0, RING, prime, 0)

    def ring_step(j, carry):
        drain(j)
        extract(j)
        fire(j + RING)
        return carry

    lax.fori_loop(0, B_PER_W - RING, ring_step, 0)

    def tail(j, carry):
        drain(j)
        extract(j)
        return carry

    lax.fori_loop(B_PER_W - RING, B_PER_W, tail, 0)

    def chunk16(jc, carry):
        ds16 = pl.ds(jc * L, L)
        acc = jnp.zeros((L,), jnp.float32)
        for k in range(EMBED_K):
            dsk = pl.ds(k * B_PER_W + jc * L, L)
            acc = acc + uflat[dsk] * vflat[dsk]
        out_v[ds16] = acc
        return carry

    lax.fori_loop(0, B_PER_W // L, chunk16, 0)

    pltpu.sync_copy(out_v, out_hbm.at[pl.ds(base, B_PER_W)])


@jax.jit
def kernel(x, W, H):
    mesh = plsc.VectorSubcoreMesh(core_axis_name="c", subcore_axis_name="s")
    f = functools.partial(
        pl.kernel,
        mesh=mesh,
        compiler_params=pltpu.CompilerParams(needs_layout_passes=False),
        out_type=jax.ShapeDtypeStruct((BATCH,), jnp.float32),
        scratch_types=[
            pltpu.VMEM((B_PER_W + L,), jnp.int32),
            pltpu.VMEM((B_PER_W + L,), jnp.int32),
            pltpu.VMEM((RING, 4, 8, 128), jnp.float32),
            pltpu.VMEM((RING, 4, 8, 128), jnp.float32),
            pltpu.VMEM((N_ELEMS,), jnp.float32),
            pltpu.VMEM((N_ELEMS,), jnp.float32),
            pltpu.VMEM((B_PER_W,), jnp.float32),
            pltpu.SemaphoreType.DMA,
        ],
    )(_body)
    return f(x[:, 0], x[:, 1], W.T, H.T)


# speedup vs baseline: 1.0287x; 1.0287x over previous
import functools

import jax
import jax.numpy as jnp
from jax import lax
from jax.experimental import pallas as pl
from jax.experimental.pallas import tpu as pltpu
from jax.experimental.pallas import tpu_sc as plsc

BATCH = 16384
EMBED_K = 32
L = 16
NW = 32
B_PER_W = BATCH // NW
N_ELEMS = B_PER_W * EMBED_K
RING = 8


def _body(u_hbm, v_hbm, wt_hbm, ht_hbm, out_hbm,
          uidx_s, vidx_s, ustage, vstage,
          uflat, vflat, out_v, sem):
    wid = lax.axis_index("s") * 2 + lax.axis_index("c")
    base = wid * B_PER_W

    wt3 = wt_hbm.reshape(4, 8, 1000000)
    ht3 = ht_hbm.reshape(4, 8, 1000000)

    pltpu.sync_copy(u_hbm.at[pl.ds(base, B_PER_W)], uidx_s.at[pl.ds(0, B_PER_W)])
    pltpu.sync_copy(v_hbm.at[pl.ds(base, B_PER_W)], vidx_s.at[pl.ds(0, B_PER_W)])

    lane = lax.iota(jnp.int32, L)
    tk_a, s_a = lane >> 3, lane & 7
    tk_b, s_b = 2 + (lane >> 3), lane & 7
    ka_base = lane * B_PER_W
    kb_base = (lane + L) * B_PER_W

    def fire(j):
        slot = j & (RING - 1)
        u = uidx_s[pl.ds(j, L)][0]
        v = vidx_s[pl.ds(j, L)][0]
        uoff = pl.multiple_of((u >> 7) * 128, 128)
        voff = pl.multiple_of((v >> 7) * 128, 128)
        pltpu.async_copy(
            wt3.at[:, :, pl.ds(uoff, 128)], ustage.at[slot], sem)
        pltpu.async_copy(
            ht3.at[:, :, pl.ds(voff, 128)], vstage.at[slot], sem)

    def drain(j):
        slot = j & (RING - 1)
        pltpu.make_async_copy(
            wt3.at[:, :, pl.ds(0, 128)], ustage.at[slot], sem).wait()
        pltpu.make_async_copy(
            ht3.at[:, :, pl.ds(0, 128)], vstage.at[slot], sem).wait()

    def extract(j):
        slot16 = jnp.full((L,), j & (RING - 1), jnp.int32)
        ucol = jnp.full((L,), uidx_s[pl.ds(j, L)][0] & 127, jnp.int32)
        vcol = jnp.full((L,), vidx_s[pl.ds(j, L)][0] & 127, jnp.int32)
        ua = plsc.load_gather(ustage, [slot16, tk_a, s_a, ucol])
        ub = plsc.load_gather(ustage, [slot16, tk_b, s_b, ucol])
        va = plsc.load_gather(vstage, [slot16, tk_a, s_a, vcol])
        vb = plsc.load_gather(vstage, [slot16, tk_b, s_b, vcol])
        plsc.store_scatter(uflat, [ka_base + j], ua)
        plsc.store_scatter(uflat, [kb_base + j], ub)
        plsc.store_scatter(vflat, [ka_base + j], va)
        plsc.store_scatter(vflat, [kb_base + j], vb)

    def prime(j, carry):
        fire(j)
        return carry

    lax.fori_loop(0, RING, prime, 0)

    def ring_step(j, carry):
        drain(j)
        extract(j)
        fire(j + RING)
        return carry

    lax.fori_loop(0, B_PER_W - RING, ring_step, 0)

    def tail(j, carry):
        drain(j)
        extract(j)
        return carry

    lax.fori_loop(B_PER_W - RING, B_PER_W, tail, 0)

    def chunk16(jc, carry):
        ds16 = pl.ds(jc * L, L)
        acc = jnp.zeros((L,), jnp.float32)
        for k in range(EMBED_K):
            dsk = pl.ds(k * B_PER_W + jc * L, L)
            acc = acc + uflat[dsk] * vflat[dsk]
        out_v[ds16] = acc
        return carry

    lax.fori_loop(0, B_PER_W // L, chunk16, 0)

    pltpu.sync_copy(out_v, out_hbm.at[pl.ds(base, B_PER_W)])


@jax.jit
def kernel(x, W, H):
    mesh = plsc.VectorSubcoreMesh(core_axis_name="c", subcore_axis_name="s")
    f = functools.partial(
        pl.kernel,
        mesh=mesh,
        compiler_params=pltpu.CompilerParams(needs_layout_passes=False),
        out_type=jax.ShapeDtypeStruct((BATCH,), jnp.float32),
        scratch_types=[
            pltpu.VMEM((B_PER_W + L,), jnp.int32),
            pltpu.VMEM((B_PER_W + L,), jnp.int32),
            pltpu.VMEM((RING, 4, 8, 128), jnp.float32),
            pltpu.VMEM((RING, 4, 8, 128), jnp.float32),
            pltpu.VMEM((N_ELEMS,), jnp.float32),
            pltpu.VMEM((N_ELEMS,), jnp.float32),
            pltpu.VMEM((B_PER_W,), jnp.float32),
            pltpu.SemaphoreType.DMA,
        ],
    )(_body)
    return f(x[:, 0], x[:, 1], W.T, H.T)


# speedup vs baseline: 1.0328x; 1.0040x over previous
import functools

import jax
import jax.numpy as jnp
from jax import lax
from jax.experimental import pallas as pl
from jax.experimental.pallas import tpu as pltpu
from jax.experimental.pallas import tpu_sc as plsc

BATCH = 16384
EMBED_K = 32
L = 16
NW = 32
B_PER_W = BATCH // NW
N_ELEMS = B_PER_W * EMBED_K
RING = 8


def _body(xt_hbm, wt_hbm, ht_hbm, out_hbm,
          xv, ustage, vstage,
          uflat, vflat, out_v, sem):
    wid = lax.axis_index("s") * 2 + lax.axis_index("c")
    base = wid * B_PER_W

    wt3 = wt_hbm.reshape(4, 8, 1000000)
    ht3 = ht_hbm.reshape(4, 8, 1000000)

    pltpu.sync_copy(xt_hbm.at[:, pl.ds(base, B_PER_W)],
                    xv.at[:, pl.ds(0, B_PER_W)])

    lane = lax.iota(jnp.int32, L)
    tk_a, s_a = lane >> 3, lane & 7
    tk_b, s_b = 2 + (lane >> 3), lane & 7
    ka_base = lane * B_PER_W
    kb_base = (lane + L) * B_PER_W

    def fire(j):
        slot = j & (RING - 1)
        u = xv[0, pl.ds(j, L)][0]
        v = xv[1, pl.ds(j, L)][0]
        uoff = pl.multiple_of((u >> 7) * 128, 128)
        voff = pl.multiple_of((v >> 7) * 128, 128)
        pltpu.async_copy(
            wt3.at[:, :, pl.ds(uoff, 128)], ustage.at[slot], sem)
        pltpu.async_copy(
            ht3.at[:, :, pl.ds(voff, 128)], vstage.at[slot], sem)

    def drain(j):
        slot = j & (RING - 1)
        pltpu.make_async_copy(
            wt3.at[:, :, pl.ds(0, 128)], ustage.at[slot], sem).wait()
        pltpu.make_async_copy(
            ht3.at[:, :, pl.ds(0, 128)], vstage.at[slot], sem).wait()

    def extract(j):
        slot16 = jnp.full((L,), j & (RING - 1), jnp.int32)
        ucol = jnp.full((L,), xv[0, pl.ds(j, L)][0] & 127, jnp.int32)
        vcol = jnp.full((L,), xv[1, pl.ds(j, L)][0] & 127, jnp.int32)
        ua = plsc.load_gather(ustage, [slot16, tk_a, s_a, ucol])
        ub = plsc.load_gather(ustage, [slot16, tk_b, s_b, ucol])
        va = plsc.load_gather(vstage, [slot16, tk_a, s_a, vcol])
        vb = plsc.load_gather(vstage, [slot16, tk_b, s_b, vcol])
        plsc.store_scatter(uflat, [ka_base + j], ua)
        plsc.store_scatter(uflat, [kb_base + j], ub)
        plsc.store_scatter(vflat, [ka_base + j], va)
        plsc.store_scatter(vflat, [kb_base + j], vb)

    def prime(j, carry):
        fire(j)
        return carry

    lax.fori_loop(0, RING, prime, 0)

    def ring_step(j, carry):
        drain(j)
        extract(j)
        fire(j + RING)
        return carry

    lax.fori_loop(0, B_PER_W - RING, ring_step, 0)

    def tail(j, carry):
        drain(j)
        extract(j)
        return carry

    lax.fori_loop(B_PER_W - RING, B_PER_W, tail, 0)

    def chunk16(jc, carry):
        ds16 = pl.ds(jc * L, L)
        acc = jnp.zeros((L,), jnp.float32)
        for k in range(EMBED_K):
            dsk = pl.ds(k * B_PER_W + jc * L, L)
            acc = acc + uflat[dsk] * vflat[dsk]
        out_v[ds16] = acc
        return carry

    lax.fori_loop(0, B_PER_W // L, chunk16, 0)

    pltpu.sync_copy(out_v, out_hbm.at[pl.ds(base, B_PER_W)])


@jax.jit
def kernel(x, W, H):
    mesh = plsc.VectorSubcoreMesh(core_axis_name="c", subcore_axis_name="s")
    f = functools.partial(
        pl.kernel,
        mesh=mesh,
        compiler_params=pltpu.CompilerParams(needs_layout_passes=False),
        out_type=jax.ShapeDtypeStruct((BATCH,), jnp.float32),
        scratch_types=[
            pltpu.VMEM((2, B_PER_W + L), jnp.int32),
            pltpu.VMEM((RING, 4, 8, 128), jnp.float32),
            pltpu.VMEM((RING, 4, 8, 128), jnp.float32),
            pltpu.VMEM((N_ELEMS,), jnp.float32),
            pltpu.VMEM((N_ELEMS,), jnp.float32),
            pltpu.VMEM((B_PER_W,), jnp.float32),
            pltpu.SemaphoreType.DMA,
        ],
    )(_body)
    return f(x.T, W.T, H.T)
